# Initial kernel scaffold; baseline (speedup 1.0000x reference)
import jax
import jax.numpy as jnp
from jax.experimental import pallas as pl


def kernel(x_text_indices, node_selectors, edge_index, edge_type, edge_positions, emb, pos_table, Wp, bp, We, be, W_ih, W_hh, b_ih, b_hh, W1, b1, W2, b2):
    raise NotImplementedError("write your pallas kernel here")


# speedup vs baseline: 1.5913x; 1.5913x over previous
import functools

import jax
import jax.numpy as jnp
from jax import lax
from jax.experimental import pallas as pl
from jax.experimental.pallas import tpu as pltpu
from jax.experimental.pallas import tpu_sc as plsc

_N = 50000
_E = 800000
_HID = 64
_NW = 32
_SC_HALF = _N // 2
_SROWS = 25024
_ROWS_PER_TILE = _SROWS // 16
_EPT = _E // 16
_KS = 80
_NCH_S = _EPT // _KS
_MSG_W = 72

_mesh = plsc.VectorSubcoreMesh(core_axis_name="c", subcore_axis_name="s")


def _make_gather(n_rows, k):
    bpw = n_rows // _NW
    nch = bpw // k

    @functools.partial(
        pl.kernel, mesh=_mesh,
        compiler_params=pltpu.CompilerParams(use_tc_tiling_on_sc=False),
        out_type=jax.ShapeDtypeStruct((n_rows, _HID), jnp.float32),
        scratch_types=[
            pltpu.VMEM((k,), jnp.int32),
            pltpu.VMEM((k, _HID), jnp.float32),
            pltpu.SemaphoreType.DMA,
        ],
    )
    def gather_k(table_hbm, idx_hbm, out_hbm, idx_v, rows_v, sem):
        wid = lax.axis_index("s") * 2 + lax.axis_index("c")
        base = wid * bpw

        def body(c, carry):
            off = base + c * k
            pltpu.sync_copy(idx_hbm.at[pl.ds(off, k)], idx_v)
            pltpu.async_copy(table_hbm.at[idx_v], rows_v, sem).wait()
            pltpu.sync_copy(rows_v, out_hbm.at[pl.ds(off, k)])
            return carry

        lax.fori_loop(0, nch, body, 0)

    return gather_k


@functools.partial(
    pl.kernel, mesh=_mesh,
    compiler_params=pltpu.CompilerParams(use_tc_tiling_on_sc=False),
    out_type=jax.ShapeDtypeStruct((2 * _SROWS, _MSG_W), jnp.float32),
    scratch_types=[
        pltpu.VMEM_SHARED((_SROWS, _MSG_W), jnp.float32),
        pltpu.VMEM((_KS,), jnp.int32),
        pltpu.VMEM((_KS,), jnp.int32),
        pltpu.VMEM((_KS, _MSG_W), jnp.float32),
    ],
)
def _scatter_k(msg_hbm, dst_hbm, zeros_hbm, out_hbm, s_sh, ibuf, dbuf, rbuf):
    c = lax.axis_index("c")
    s = lax.axis_index("s")
    base = c * _SC_HALF

    pltpu.sync_copy(zeros_hbm, rbuf)
    row0 = s * _ROWS_PER_TILE

    def zb(i, carry):
        pltpu.sync_copy(rbuf, s_sh.at[pl.ds(row0 + i * _KS, _KS)])
        return carry

    lax.fori_loop(0, 19, zb, 0)
    pltpu.sync_copy(rbuf.at[pl.ds(0, 44)], s_sh.at[pl.ds(row0 + 1520, 44)])
    plsc.subcore_barrier()

    def body(ch, carry):
        eoff = s * _EPT + ch * _KS
        pltpu.sync_copy(dst_hbm.at[pl.ds(eoff, _KS)], dbuf)
        for i in range(_KS // 16):
            v = dbuf[pl.ds(i * 16, 16)]
            loc = v - base
            ok = (loc >= 0) & (loc < _SC_HALF)
            ibuf[pl.ds(i * 16, 16)] = jnp.where(ok, loc, _SC_HALF)
        pltpu.sync_copy(msg_hbm.at[pl.ds(eoff, _KS)], rbuf)
        pltpu.sync_copy(rbuf, s_sh.at[ibuf], add=True)
        return carry

    lax.fori_loop(0, _NCH_S, body, 0)
    plsc.subcore_barrier()

    out0 = c * _SROWS + row0

    def wb(i, carry):
        pltpu.sync_copy(s_sh.at[pl.ds(row0 + i * _KS, _KS)],
                        out_hbm.at[pl.ds(out0 + i * _KS, _KS)])
        return carry

    lax.fori_loop(0, 19, wb, 0)
    pltpu.sync_copy(s_sh.at[pl.ds(row0 + 1520, 44)],
                    out_hbm.at[pl.ds(out0 + 1520, 44)])


_BE = 3200


def _msg_body(xj_ref, posf_ref, typef_ref, we_ref, be_ref, pt_ref, wp_ref,
              bp_ref, out_ref):
    xj = xj_ref[...]
    posf = posf_ref[...]
    typef = typef_ref[...]
    pt = pt_ref[...]
    gt = 2.0 * jax.nn.sigmoid(
        jnp.dot(pt, wp_ref[...], preferred_element_type=jnp.float32)
        + bp_ref[...])
    ids = lax.broadcasted_iota(jnp.int32, (1, 72), 1).astype(jnp.float32)
    onehot = (posf == ids).astype(jnp.float32)
    gate = jnp.dot(onehot, gt, preferred_element_type=jnp.float32)
    g = xj * gate
    be = be_ref[...]
    acc = jnp.zeros_like(g)
    for t in range(3):
        mt = (typef == float(t)).astype(jnp.float32)
        acc = acc + mt * (
            jnp.dot(g, we_ref[t], preferred_element_type=jnp.float32) + be[t])
    out_ref[:, :_HID] = acc
    out_ref[:, _HID:_HID + 1] = jnp.ones((_BE, 1), jnp.float32)
    out_ref[:, _HID + 1:] = jnp.zeros((_BE, _MSG_W - _HID - 1), jnp.float32)


def _tc_messages(xj, posf, typef, we, be, pt_pad, wp, bp):
    nb = _E // _BE
    return pl.pallas_call(
        _msg_body,
        grid=(nb,),
        in_specs=[
            pl.BlockSpec((_BE, _HID), lambda i: (i, 0)),
            pl.BlockSpec((_BE, 1), lambda i: (i, 0)),
            pl.BlockSpec((_BE, 1), lambda i: (i, 0)),
            pl.BlockSpec((3, _HID, _HID), lambda i: (0, 0, 0)),
            pl.BlockSpec((3, _HID), lambda i: (0, 0)),
            pl.BlockSpec((72, 32), lambda i: (0, 0)),
            pl.BlockSpec((32, _HID), lambda i: (0, 0)),
            pl.BlockSpec((1, _HID), lambda i: (0, 0)),
        ],
        out_specs=pl.BlockSpec((_BE, _MSG_W), lambda i: (i, 0)),
        out_shape=jax.ShapeDtypeStruct((_E, _MSG_W), jnp.float32),
    )(xj, posf, typef, we, be, pt_pad, wp, bp)


_BN = 2000


def _gru_body(sums_ref, cnt_ref, h_ref, wiht_ref, whht_ref, bih_ref, bhh_ref,
              out_ref):
    cnt = jnp.maximum(cnt_ref[...], 1.0)
    agg = sums_ref[...] / cnt
    h = h_ref[...]
    gi = jnp.dot(agg, wiht_ref[...], preferred_element_type=jnp.float32) \
        + bih_ref[...]
    gh = jnp.dot(h, whht_ref[...], preferred_element_type=jnp.float32) \
        + bhh_ref[...]
    r = jax.nn.sigmoid(gi[:, :64] + gh[:, :64])
    z = jax.nn.sigmoid(gi[:, 64:128] + gh[:, 64:128])
    n = jnp.tanh(gi[:, 128:] + r * gh[:, 128:])
    out_ref[...] = (1.0 - z) * n + z * h


def _tc_gru(sums, cnt, h, wiht, whht, bih, bhh):
    nb = _N // _BN
    return pl.pallas_call(
        _gru_body,
        grid=(nb,),
        in_specs=[
            pl.BlockSpec((_BN, _HID), lambda i: (i, 0)),
            pl.BlockSpec((_BN, 1), lambda i: (i, 0)),
            pl.BlockSpec((_BN, _HID), lambda i: (i, 0)),
            pl.BlockSpec((_HID, 3 * _HID), lambda i: (0, 0)),
            pl.BlockSpec((_HID, 3 * _HID), lambda i: (0, 0)),
            pl.BlockSpec((1, 3 * _HID), lambda i: (0, 0)),
            pl.BlockSpec((1, 3 * _HID), lambda i: (0, 0)),
        ],
        out_specs=pl.BlockSpec((_BN, _HID), lambda i: (i, 0)),
        out_shape=jax.ShapeDtypeStruct((_N, _HID), jnp.float32),
    )(sums, cnt, h, wiht, whht, bih, bhh)


def _readout_body(h_ref, h0_ref, w1_ref, b1_ref, w2_ref, b2_ref, out_ref):
    comb = jnp.concatenate([h_ref[...], h0_ref[...]], axis=1)
    a = jax.nn.relu(
        jnp.dot(comb, w1_ref[...], preferred_element_type=jnp.float32)
        + b1_ref[...])
    out_ref[...] = jnp.dot(a, w2_ref[...],
                           preferred_element_type=jnp.float32) + b2_ref[...]


def _tc_readout(h, h0, w1, b1, w2, b2):
    nb = _N // _BN
    return pl.pallas_call(
        _readout_body,
        grid=(nb,),
        in_specs=[
            pl.BlockSpec((_BN, _HID), lambda i: (i, 0)),
            pl.BlockSpec((_BN, _HID), lambda i: (i, 0)),
            pl.BlockSpec((2 * _HID, _HID), lambda i: (0, 0)),
            pl.BlockSpec((1, _HID), lambda i: (0, 0)),
            pl.BlockSpec((_HID, 1), lambda i: (0, 0)),
            pl.BlockSpec((1, 1), lambda i: (0, 0)),
        ],
        out_specs=pl.BlockSpec((_BN, 1), lambda i: (i, 0)),
        out_shape=jax.ShapeDtypeStruct((_N, 1), jnp.float32),
    )(h, h0, w1, b1, w2, b2)


_gather_emb = _make_gather(50176, 112)
_gather_h = _make_gather(_E, 40)


def kernel(x_text_indices, node_selectors, edge_index, edge_type,
           edge_positions, emb, pos_table, Wp, bp, We, be, W_ih, W_hh,
           b_ih, b_hh, W1, b1, W2, b2):
    f32 = jnp.float32
    src = edge_index[0].astype(jnp.int32)
    dst = edge_index[1].astype(jnp.int32)
    posf = edge_positions.astype(f32).reshape(_E, 1)
    typef = edge_type.astype(f32).reshape(_E, 1)

    emb_pad = jnp.concatenate([emb, jnp.zeros((emb.shape[0], 2), f32)], axis=1)
    xt_pad = jnp.concatenate(
        [x_text_indices.astype(jnp.int32), jnp.zeros((176,), jnp.int32)])
    text = _gather_emb(emb_pad, xt_pad)
    h0 = jnp.concatenate([text[:_N, :62], node_selectors], axis=1)

    pt_pad = jnp.concatenate([pos_table, jnp.zeros((7, 32), f32)], axis=0)
    bp2 = bp.reshape(1, _HID)
    wiht = W_ih.T
    whht = W_hh.T
    bih2 = b_ih.reshape(1, 3 * _HID)
    bhh2 = b_hh.reshape(1, 3 * _HID)
    zeros_tile = jnp.zeros((_KS, _MSG_W), f32)

    h = h0
    for _it in range(2):
        xj = _gather_h(h, src)
        msgs = _tc_messages(xj, posf, typef, We, be, pt_pad, Wp, bp2)
        seg = _scatter_k(msgs, dst, zeros_tile)
        sums = jnp.concatenate(
            [seg[:_SC_HALF, :_HID], seg[_SROWS:_SROWS + _SC_HALF, :_HID]])
        cnt = jnp.concatenate(
            [seg[:_SC_HALF, _HID:_HID + 1],
             seg[_SROWS:_SROWS + _SC_HALF, _HID:_HID + 1]])
        h = _tc_gru(sums, cnt, h, wiht, whht, bih2, bhh2)

    return _tc_readout(h, h0, W1, b1.reshape(1, _HID), W2, b2.reshape(1, 1))


# speedup vs baseline: 2.0141x; 1.2656x over previous
import functools

import jax
import jax.numpy as jnp
from jax import lax
from jax.experimental import pallas as pl
from jax.experimental.pallas import tpu as pltpu
from jax.experimental.pallas import tpu_sc as plsc

_N = 50000
_E = 800000
_HID = 64
_NW = 32
_SC_HALF = _N // 2
_SROWS = 25024
_ROWS_PER_TILE = _SROWS // 16
_EPT = _E // 16
_KS = 80
_KG = 160
_NCH_S = 312
_MSG_W = 72

_mesh = plsc.VectorSubcoreMesh(core_axis_name="c", subcore_axis_name="s")


def _make_gather(n_rows, k_sub, n_sub, n_groups, tail):
    bpw = n_rows // _NW
    grp = k_sub * n_sub
    assert n_groups * grp + tail == bpw

    @functools.partial(
        pl.kernel, mesh=_mesh,
        compiler_params=pltpu.CompilerParams(use_tc_tiling_on_sc=False),
        out_type=jax.ShapeDtypeStruct((n_rows, _HID), jnp.float32),
        scratch_types=[
            pltpu.VMEM((grp,), jnp.int32),
            pltpu.VMEM((grp, _HID), jnp.float32),
            pltpu.SemaphoreType.DMA,
        ],
    )
    def gather_k(table_hbm, idx_hbm, out_hbm, idx_v, rows_v, sem):
        wid = lax.axis_index("s") * 2 + lax.axis_index("c")
        base = wid * bpw

        def body(c, carry):
            off = base + c * grp
            pltpu.sync_copy(idx_hbm.at[pl.ds(off, grp)], idx_v)
            cps = [
                pltpu.async_copy(
                    table_hbm.at[idx_v.at[pl.ds(j * k_sub, k_sub)]],
                    rows_v.at[pl.ds(j * k_sub, k_sub)], sem)
                for j in range(n_sub)
            ]
            for cp in cps:
                cp.wait()
            pltpu.sync_copy(rows_v, out_hbm.at[pl.ds(off, grp)])
            return carry

        lax.fori_loop(0, n_groups, body, 0)

        if tail:
            off = base + n_groups * grp
            pltpu.sync_copy(idx_hbm.at[pl.ds(off, tail)],
                            idx_v.at[pl.ds(0, tail)])
            pltpu.async_copy(table_hbm.at[idx_v.at[pl.ds(0, tail)]],
                             rows_v.at[pl.ds(0, tail)], sem).wait()
            pltpu.sync_copy(rows_v.at[pl.ds(0, tail)],
                            out_hbm.at[pl.ds(off, tail)])

    return gather_k


@functools.partial(
    pl.kernel, mesh=_mesh,
    compiler_params=pltpu.CompilerParams(use_tc_tiling_on_sc=False),
    out_type=jax.ShapeDtypeStruct((2 * _SROWS, _MSG_W), jnp.float32),
    scratch_types=[
        pltpu.VMEM_SHARED((_SROWS, _MSG_W), jnp.float32),
        pltpu.VMEM((2, _KS), jnp.int32),
        pltpu.VMEM((_KG,), jnp.int32),
        pltpu.VMEM((_KG, _MSG_W), jnp.float32),
        pltpu.SemaphoreType.DMA,
    ],
)
def _scatter_k(msg_hbm, dst_hbm, zeros_hbm, out_hbm, s_sh, ibuf, dbuf, rbuf,
               sem):
    c = lax.axis_index("c")
    s = lax.axis_index("s")
    base = c * _SC_HALF

    pltpu.sync_copy(zeros_hbm, rbuf)
    row0 = s * _ROWS_PER_TILE

    def zb(i, carry):
        pltpu.sync_copy(rbuf, s_sh.at[pl.ds(row0 + i * _KG, _KG)])
        return carry

    lax.fori_loop(0, 9, zb, 0)
    pltpu.sync_copy(rbuf.at[pl.ds(0, 124)], s_sh.at[pl.ds(row0 + 1440, 124)])
    plsc.subcore_barrier()

    def _mk_idx(i):
        v = dbuf[pl.ds(i * 16, 16)]
        loc = v - base
        ok = (loc >= 0) & (loc < _SC_HALF)
        return jnp.where(ok, loc, _SC_HALF)

    def body(g, carry):
        eoff = s * _EPT + g * _KG
        pltpu.sync_copy(dst_hbm.at[pl.ds(eoff, _KG)], dbuf)
        mc = pltpu.async_copy(msg_hbm.at[pl.ds(eoff, _KG)], rbuf, sem)
        for j in range(2):
            for i in range(_KS // 16):
                ibuf[j, pl.ds(i * 16, 16)] = _mk_idx(j * (_KS // 16) + i)
        mc.wait()
        for j in range(2):
            pltpu.sync_copy(rbuf.at[pl.ds(j * _KS, _KS)],
                            s_sh.at[ibuf.at[j]], add=True)
        return carry

    lax.fori_loop(0, _NCH_S, body, 0)

    eoff = s * _EPT + _NCH_S * _KG
    pltpu.sync_copy(dst_hbm.at[pl.ds(eoff, _KS)], dbuf.at[pl.ds(0, _KS)])
    for i in range(_KS // 16):
        ibuf[0, pl.ds(i * 16, 16)] = _mk_idx(i)
    pltpu.sync_copy(msg_hbm.at[pl.ds(eoff, _KS)], rbuf.at[pl.ds(0, _KS)])
    pltpu.sync_copy(rbuf.at[pl.ds(0, _KS)], s_sh.at[ibuf.at[0]], add=True)

    plsc.subcore_barrier()

    out0 = c * _SROWS + row0

    def wb(i, carry):
        pltpu.sync_copy(s_sh.at[pl.ds(row0 + i * _KG, _KG)],
                        out_hbm.at[pl.ds(out0 + i * _KG, _KG)])
        return carry

    lax.fori_loop(0, 9, wb, 0)
    pltpu.sync_copy(s_sh.at[pl.ds(row0 + 1440, 124)],
                    out_hbm.at[pl.ds(out0 + 1440, 124)])


_BE = 3200


def _msg_body(xj_ref, posf_ref, typef_ref, we_ref, be_ref, pt_ref, wp_ref,
              bp_ref, out_ref):
    xj = xj_ref[...]
    posf = posf_ref[...]
    typef = typef_ref[...]
    pt = pt_ref[...]
    gt = 2.0 * jax.nn.sigmoid(
        jnp.dot(pt, wp_ref[...], preferred_element_type=jnp.float32)
        + bp_ref[...])
    ids = lax.broadcasted_iota(jnp.int32, (1, 72), 1).astype(jnp.float32)
    onehot = (posf == ids).astype(jnp.float32)
    gate = jnp.dot(onehot, gt, preferred_element_type=jnp.float32)
    g = xj * gate
    be = be_ref[...]
    acc = jnp.zeros_like(g)
    for t in range(3):
        mt = (typef == float(t)).astype(jnp.float32)
        acc = acc + mt * (
            jnp.dot(g, we_ref[t], preferred_element_type=jnp.float32) + be[t])
    out_ref[:, :_HID] = acc
    out_ref[:, _HID:_HID + 1] = jnp.ones((_BE, 1), jnp.float32)
    out_ref[:, _HID + 1:] = jnp.zeros((_BE, _MSG_W - _HID - 1), jnp.float32)


def _tc_messages(xj, posf, typef, we, be, pt_pad, wp, bp):
    nb = _E // _BE
    return pl.pallas_call(
        _msg_body,
        grid=(nb,),
        in_specs=[
            pl.BlockSpec((_BE, _HID), lambda i: (i, 0)),
            pl.BlockSpec((_BE, 1), lambda i: (i, 0)),
            pl.BlockSpec((_BE, 1), lambda i: (i, 0)),
            pl.BlockSpec((3, _HID, _HID), lambda i: (0, 0, 0)),
            pl.BlockSpec((3, _HID), lambda i: (0, 0)),
            pl.BlockSpec((72, 32), lambda i: (0, 0)),
            pl.BlockSpec((32, _HID), lambda i: (0, 0)),
            pl.BlockSpec((1, _HID), lambda i: (0, 0)),
        ],
        out_specs=pl.BlockSpec((_BE, _MSG_W), lambda i: (i, 0)),
        out_shape=jax.ShapeDtypeStruct((_E, _MSG_W), jnp.float32),
    )(xj, posf, typef, we, be, pt_pad, wp, bp)


_BN = 2000


def _gru_body(sums_ref, cnt_ref, h_ref, wiht_ref, whht_ref, bih_ref, bhh_ref,
              out_ref):
    cnt = jnp.maximum(cnt_ref[...], 1.0)
    agg = sums_ref[...] / cnt
    h = h_ref[...]
    gi = jnp.dot(agg, wiht_ref[...], preferred_element_type=jnp.float32) \
        + bih_ref[...]
    gh = jnp.dot(h, whht_ref[...], preferred_element_type=jnp.float32) \
        + bhh_ref[...]
    r = jax.nn.sigmoid(gi[:, :64] + gh[:, :64])
    z = jax.nn.sigmoid(gi[:, 64:128] + gh[:, 64:128])
    n = jnp.tanh(gi[:, 128:] + r * gh[:, 128:])
    out_ref[...] = (1.0 - z) * n + z * h


def _tc_gru(sums, cnt, h, wiht, whht, bih, bhh):
    nb = _N // _BN
    return pl.pallas_call(
        _gru_body,
        grid=(nb,),
        in_specs=[
            pl.BlockSpec((_BN, _HID), lambda i: (i, 0)),
            pl.BlockSpec((_BN, 1), lambda i: (i, 0)),
            pl.BlockSpec((_BN, _HID), lambda i: (i, 0)),
            pl.BlockSpec((_HID, 3 * _HID), lambda i: (0, 0)),
            pl.BlockSpec((_HID, 3 * _HID), lambda i: (0, 0)),
            pl.BlockSpec((1, 3 * _HID), lambda i: (0, 0)),
            pl.BlockSpec((1, 3 * _HID), lambda i: (0, 0)),
        ],
        out_specs=pl.BlockSpec((_BN, _HID), lambda i: (i, 0)),
        out_shape=jax.ShapeDtypeStruct((_N, _HID), jnp.float32),
    )(sums, cnt, h, wiht, whht, bih, bhh)


def _readout_body(h_ref, h0_ref, w1_ref, b1_ref, w2_ref, b2_ref, out_ref):
    comb = jnp.concatenate([h_ref[...], h0_ref[...]], axis=1)
    a = jax.nn.relu(
        jnp.dot(comb, w1_ref[...], preferred_element_type=jnp.float32)
        + b1_ref[...])
    out_ref[...] = jnp.dot(a, w2_ref[...],
                           preferred_element_type=jnp.float32) + b2_ref[...]


def _tc_readout(h, h0, w1, b1, w2, b2):
    nb = _N // _BN
    return pl.pallas_call(
        _readout_body,
        grid=(nb,),
        in_specs=[
            pl.BlockSpec((_BN, _HID), lambda i: (i, 0)),
            pl.BlockSpec((_BN, _HID), lambda i: (i, 0)),
            pl.BlockSpec((2 * _HID, _HID), lambda i: (0, 0)),
            pl.BlockSpec((1, _HID), lambda i: (0, 0)),
            pl.BlockSpec((_HID, 1), lambda i: (0, 0)),
            pl.BlockSpec((1, 1), lambda i: (0, 0)),
        ],
        out_specs=pl.BlockSpec((_BN, 1), lambda i: (i, 0)),
        out_shape=jax.ShapeDtypeStruct((_N, 1), jnp.float32),
    )(h, h0, w1, b1, w2, b2)


_gather_emb = _make_gather(50176, 112, 2, 7, 0)
_gather_h = _make_gather(_E, 40, 16, 39, 40)


def kernel(x_text_indices, node_selectors, edge_index, edge_type,
           edge_positions, emb, pos_table, Wp, bp, We, be, W_ih, W_hh,
           b_ih, b_hh, W1, b1, W2, b2):
    f32 = jnp.float32
    src = edge_index[0].astype(jnp.int32)
    dst = edge_index[1].astype(jnp.int32)
    posf = edge_positions.astype(f32).reshape(_E, 1)
    typef = edge_type.astype(f32).reshape(_E, 1)

    emb_pad = jnp.concatenate([emb, jnp.zeros((emb.shape[0], 2), f32)], axis=1)
    xt_pad = jnp.concatenate(
        [x_text_indices.astype(jnp.int32), jnp.zeros((176,), jnp.int32)])
    text = _gather_emb(emb_pad, xt_pad)
    h0 = jnp.concatenate([text[:_N, :62], node_selectors], axis=1)

    pt_pad = jnp.concatenate([pos_table, jnp.zeros((7, 32), f32)], axis=0)
    bp2 = bp.reshape(1, _HID)
    wiht = W_ih.T
    whht = W_hh.T
    bih2 = b_ih.reshape(1, 3 * _HID)
    bhh2 = b_hh.reshape(1, 3 * _HID)
    zeros_tile = jnp.zeros((_KG, _MSG_W), f32)

    h = h0
    for _it in range(2):
        xj = _gather_h(h, src)
        msgs = _tc_messages(xj, posf, typef, We, be, pt_pad, Wp, bp2)
        seg = _scatter_k(msgs, dst, zeros_tile)
        sums = jnp.concatenate(
            [seg[:_SC_HALF, :_HID], seg[_SROWS:_SROWS + _SC_HALF, :_HID]])
        cnt = jnp.concatenate(
            [seg[:_SC_HALF, _HID:_HID + 1],
             seg[_SROWS:_SROWS + _SC_HALF, _HID:_HID + 1]])
        h = _tc_gru(sums, cnt, h, wiht, whht, bih2, bhh2)

    return _tc_readout(h, h0, W1, b1.reshape(1, _HID), W2, b2.reshape(1, 1))


# speedup vs baseline: 2.0756x; 1.0305x over previous
import functools

import jax
import jax.numpy as jnp
from jax import lax
from jax.experimental import pallas as pl
from jax.experimental.pallas import tpu as pltpu
from jax.experimental.pallas import tpu_sc as plsc

_N = 50000
_E = 800000
_HID = 64
_NW = 32
_SC_HALF = _N // 2
_SROWS = 25024
_ROWS_PER_TILE = _SROWS // 16
_EPT = _E // 16
_KS = 112
_KG = 224
_NCH_S = 223
_KT = 48
_MSG_W = 72

_mesh = plsc.VectorSubcoreMesh(core_axis_name="c", subcore_axis_name="s")


def _make_gather(n_rows, k_sub, n_sub, n_groups, tail):
    bpw = n_rows // _NW
    grp = k_sub * n_sub
    assert n_groups * grp + tail == bpw

    @functools.partial(
        pl.kernel, mesh=_mesh,
        compiler_params=pltpu.CompilerParams(use_tc_tiling_on_sc=False),
        out_type=jax.ShapeDtypeStruct((n_rows, _HID), jnp.float32),
        scratch_types=[
            pltpu.VMEM((grp,), jnp.int32),
            pltpu.VMEM((grp, _HID), jnp.float32),
            pltpu.SemaphoreType.DMA,
        ],
    )
    def gather_k(table_hbm, idx_hbm, out_hbm, idx_v, rows_v, sem):
        wid = lax.axis_index("s") * 2 + lax.axis_index("c")
        base = wid * bpw

        def body(c, carry):
            off = base + c * grp
            pltpu.sync_copy(idx_hbm.at[pl.ds(off, grp)], idx_v)
            cps = [
                pltpu.async_copy(
                    table_hbm.at[idx_v.at[pl.ds(j * k_sub, k_sub)]],
                    rows_v.at[pl.ds(j * k_sub, k_sub)], sem)
                for j in range(n_sub)
            ]
            for cp in cps:
                cp.wait()
            pltpu.sync_copy(rows_v, out_hbm.at[pl.ds(off, grp)])
            return carry

        lax.fori_loop(0, n_groups, body, 0)

        if tail:
            off = base + n_groups * grp
            pltpu.sync_copy(idx_hbm.at[pl.ds(off, tail)],
                            idx_v.at[pl.ds(0, tail)])
            pltpu.async_copy(table_hbm.at[idx_v.at[pl.ds(0, tail)]],
                             rows_v.at[pl.ds(0, tail)], sem).wait()
            pltpu.sync_copy(rows_v.at[pl.ds(0, tail)],
                            out_hbm.at[pl.ds(off, tail)])

    return gather_k


@functools.partial(
    pl.kernel, mesh=_mesh,
    compiler_params=pltpu.CompilerParams(use_tc_tiling_on_sc=False),
    out_type=jax.ShapeDtypeStruct((2 * _SROWS, _MSG_W), jnp.float32),
    scratch_types=[
        pltpu.VMEM_SHARED((_SROWS, _MSG_W), jnp.float32),
        pltpu.VMEM((2, _KS), jnp.int32),
        pltpu.VMEM((_KG,), jnp.int32),
        pltpu.VMEM((_KG, _MSG_W), jnp.float32),
        pltpu.VMEM((1, _KT), jnp.int32),
        pltpu.SemaphoreType.DMA,
    ],
)
def _scatter_k(msg_hbm, dst_hbm, zeros_hbm, out_hbm, s_sh, ibuf, dbuf, rbuf,
               tbuf, sem):
    c = lax.axis_index("c")
    s = lax.axis_index("s")
    base = c * _SC_HALF

    pltpu.sync_copy(zeros_hbm, rbuf)
    row0 = s * _ROWS_PER_TILE

    def zb(i, carry):
        pltpu.sync_copy(rbuf, s_sh.at[pl.ds(row0 + i * _KG, _KG)])
        return carry

    lax.fori_loop(0, 6, zb, 0)
    pltpu.sync_copy(rbuf.at[pl.ds(0, 220)], s_sh.at[pl.ds(row0 + 1344, 220)])
    plsc.subcore_barrier()

    def _mk_idx(i):
        v = dbuf[pl.ds(i * 16, 16)]
        loc = v - base
        ok = (loc >= 0) & (loc < _SC_HALF)
        return jnp.where(ok, loc, _SC_HALF)

    def body(g, carry):
        eoff = s * _EPT + g * _KG
        pltpu.sync_copy(dst_hbm.at[pl.ds(eoff, _KG)], dbuf)
        mc = pltpu.async_copy(msg_hbm.at[pl.ds(eoff, _KG)], rbuf, sem)
        for j in range(2):
            for i in range(_KS // 16):
                ibuf[j, pl.ds(i * 16, 16)] = _mk_idx(j * (_KS // 16) + i)
        mc.wait()
        for j in range(2):
            pltpu.sync_copy(rbuf.at[pl.ds(j * _KS, _KS)],
                            s_sh.at[ibuf.at[j]], add=True)
        return carry

    lax.fori_loop(0, _NCH_S, body, 0)

    eoff = s * _EPT + _NCH_S * _KG
    pltpu.sync_copy(dst_hbm.at[pl.ds(eoff, _KT)], dbuf.at[pl.ds(0, _KT)])
    for i in range(_KT // 16):
        tbuf[0, pl.ds(i * 16, 16)] = _mk_idx(i)
    pltpu.sync_copy(msg_hbm.at[pl.ds(eoff, _KT)], rbuf.at[pl.ds(0, _KT)])
    pltpu.sync_copy(rbuf.at[pl.ds(0, _KT)], s_sh.at[tbuf.at[0]], add=True)

    plsc.subcore_barrier()

    out0 = c * _SROWS + row0

    def wb(i, carry):
        pltpu.sync_copy(s_sh.at[pl.ds(row0 + i * _KG, _KG)],
                        out_hbm.at[pl.ds(out0 + i * _KG, _KG)])
        return carry

    lax.fori_loop(0, 6, wb, 0)
    pltpu.sync_copy(s_sh.at[pl.ds(row0 + 1344, 220)],
                    out_hbm.at[pl.ds(out0 + 1344, 220)])


_BE = 3200


def _msg_body(xj_ref, posf_ref, typef_ref, we_ref, be_ref, pt_ref, wp_ref,
              bp_ref, out_ref):
    xj = xj_ref[...]
    posf = posf_ref[...]
    typef = typef_ref[...]
    pt = pt_ref[...]
    gt = 2.0 * jax.nn.sigmoid(
        jnp.dot(pt, wp_ref[...], preferred_element_type=jnp.float32)
        + bp_ref[...])
    ids = lax.broadcasted_iota(jnp.int32, (1, 72), 1).astype(jnp.float32)
    onehot = (posf == ids).astype(jnp.float32)
    gate = jnp.dot(onehot, gt, preferred_element_type=jnp.float32)
    g = xj * gate
    be = be_ref[...]
    acc = jnp.zeros_like(g)
    for t in range(3):
        mt = (typef == float(t)).astype(jnp.float32)
        acc = acc + mt * (
            jnp.dot(g, we_ref[t], preferred_element_type=jnp.float32) + be[t])
    out_ref[:, :_HID] = acc
    out_ref[:, _HID:_HID + 1] = jnp.ones((_BE, 1), jnp.float32)
    out_ref[:, _HID + 1:] = jnp.zeros((_BE, _MSG_W - _HID - 1), jnp.float32)


def _tc_messages(xj, posf, typef, we, be, pt_pad, wp, bp):
    nb = _E // _BE
    return pl.pallas_call(
        _msg_body,
        grid=(nb,),
        in_specs=[
            pl.BlockSpec((_BE, _HID), lambda i: (i, 0)),
            pl.BlockSpec((_BE, 1), lambda i: (i, 0)),
            pl.BlockSpec((_BE, 1), lambda i: (i, 0)),
            pl.BlockSpec((3, _HID, _HID), lambda i: (0, 0, 0)),
            pl.BlockSpec((3, _HID), lambda i: (0, 0)),
            pl.BlockSpec((72, 32), lambda i: (0, 0)),
            pl.BlockSpec((32, _HID), lambda i: (0, 0)),
            pl.BlockSpec((1, _HID), lambda i: (0, 0)),
        ],
        out_specs=pl.BlockSpec((_BE, _MSG_W), lambda i: (i, 0)),
        out_shape=jax.ShapeDtypeStruct((_E, _MSG_W), jnp.float32),
    )(xj, posf, typef, we, be, pt_pad, wp, bp)


_BN = 2000


def _gru_body(sums_ref, cnt_ref, h_ref, wiht_ref, whht_ref, bih_ref, bhh_ref,
              out_ref):
    cnt = jnp.maximum(cnt_ref[...], 1.0)
    agg = sums_ref[...] / cnt
    h = h_ref[...]
    gi = jnp.dot(agg, wiht_ref[...], preferred_element_type=jnp.float32) \
        + bih_ref[...]
    gh = jnp.dot(h, whht_ref[...], preferred_element_type=jnp.float32) \
        + bhh_ref[...]
    r = jax.nn.sigmoid(gi[:, :64] + gh[:, :64])
    z = jax.nn.sigmoid(gi[:, 64:128] + gh[:, 64:128])
    n = jnp.tanh(gi[:, 128:] + r * gh[:, 128:])
    out_ref[...] = (1.0 - z) * n + z * h


def _tc_gru(sums, cnt, h, wiht, whht, bih, bhh):
    nb = _N // _BN
    return pl.pallas_call(
        _gru_body,
        grid=(nb,),
        in_specs=[
            pl.BlockSpec((_BN, _HID), lambda i: (i, 0)),
            pl.BlockSpec((_BN, 1), lambda i: (i, 0)),
            pl.BlockSpec((_BN, _HID), lambda i: (i, 0)),
            pl.BlockSpec((_HID, 3 * _HID), lambda i: (0, 0)),
            pl.BlockSpec((_HID, 3 * _HID), lambda i: (0, 0)),
            pl.BlockSpec((1, 3 * _HID), lambda i: (0, 0)),
            pl.BlockSpec((1, 3 * _HID), lambda i: (0, 0)),
        ],
        out_specs=pl.BlockSpec((_BN, _HID), lambda i: (i, 0)),
        out_shape=jax.ShapeDtypeStruct((_N, _HID), jnp.float32),
    )(sums, cnt, h, wiht, whht, bih, bhh)


def _readout_body(h_ref, h0_ref, w1_ref, b1_ref, w2_ref, b2_ref, out_ref):
    comb = jnp.concatenate([h_ref[...], h0_ref[...]], axis=1)
    a = jax.nn.relu(
        jnp.dot(comb, w1_ref[...], preferred_element_type=jnp.float32)
        + b1_ref[...])
    out_ref[...] = jnp.dot(a, w2_ref[...],
                           preferred_element_type=jnp.float32) + b2_ref[...]


def _tc_readout(h, h0, w1, b1, w2, b2):
    nb = _N // _BN
    return pl.pallas_call(
        _readout_body,
        grid=(nb,),
        in_specs=[
            pl.BlockSpec((_BN, _HID), lambda i: (i, 0)),
            pl.BlockSpec((_BN, _HID), lambda i: (i, 0)),
            pl.BlockSpec((2 * _HID, _HID), lambda i: (0, 0)),
            pl.BlockSpec((1, _HID), lambda i: (0, 0)),
            pl.BlockSpec((_HID, 1), lambda i: (0, 0)),
            pl.BlockSpec((1, 1), lambda i: (0, 0)),
        ],
        out_specs=pl.BlockSpec((_BN, 1), lambda i: (i, 0)),
        out_shape=jax.ShapeDtypeStruct((_N, 1), jnp.float32),
    )(h, h0, w1, b1, w2, b2)


_gather_emb = _make_gather(50176, 112, 2, 7, 0)
_gather_h = _make_gather(_E, 40, 16, 39, 40)


def kernel(x_text_indices, node_selectors, edge_index, edge_type,
           edge_positions, emb, pos_table, Wp, bp, We, be, W_ih, W_hh,
           b_ih, b_hh, W1, b1, W2, b2):
    f32 = jnp.float32
    src = edge_index[0].astype(jnp.int32)
    dst = edge_index[1].astype(jnp.int32)
    posf = edge_positions.astype(f32).reshape(_E, 1)
    typef = edge_type.astype(f32).reshape(_E, 1)

    emb_pad = jnp.concatenate([emb, jnp.zeros((emb.shape[0], 2), f32)], axis=1)
    xt_pad = jnp.concatenate(
        [x_text_indices.astype(jnp.int32), jnp.zeros((176,), jnp.int32)])
    text = _gather_emb(emb_pad, xt_pad)
    h0 = jnp.concatenate([text[:_N, :62], node_selectors], axis=1)

    pt_pad = jnp.concatenate([pos_table, jnp.zeros((7, 32), f32)], axis=0)
    bp2 = bp.reshape(1, _HID)
    wiht = W_ih.T
    whht = W_hh.T
    bih2 = b_ih.reshape(1, 3 * _HID)
    bhh2 = b_hh.reshape(1, 3 * _HID)
    zeros_tile = jnp.zeros((_KG, _MSG_W), f32)

    h = h0
    for _it in range(2):
        xj = _gather_h(h, src)
        msgs = _tc_messages(xj, posf, typef, We, be, pt_pad, Wp, bp2)
        seg = _scatter_k(msgs, dst, zeros_tile)
        sums = jnp.concatenate(
            [seg[:_SC_HALF, :_HID], seg[_SROWS:_SROWS + _SC_HALF, :_HID]])
        cnt = jnp.concatenate(
            [seg[:_SC_HALF, _HID:_HID + 1],
             seg[_SROWS:_SROWS + _SC_HALF, _HID:_HID + 1]])
        h = _tc_gru(sums, cnt, h, wiht, whht, bih2, bhh2)

    return _tc_readout(h, h0, W1, b1.reshape(1, _HID), W2, b2.reshape(1, 1))
